# baseline (speedup 1.0000x reference)
import jax
import jax.numpy as jnp
from jax.experimental import pallas as pl


def _conv(x, W, b, ei, sel, itp):
    y = jnp.einsum('nc,sco->nso', x, W)
    msgs = y[ei[0], sel] * itp[:, None]
    return jax.ops.segment_sum(msgs, ei[1], num_segments=x.shape[0]) + b


def _bias_add_kernel(x_ref, b_ref, o_ref):
    o_ref[...] = x_ref[...] + b_ref[...]


def kernel(x, edge_index_3, selections_3, interps_3, edge_index_2, selections_2, interps_2, edge_index_1, selections_1, interps_1, edge_index_0, selections_0, interps_0, clusters_2, clusters_1, clusters_0, W11, b11, W12, b12, W13, b13, W14, b14, W15, b15, W16, b16, W17, b17, W18, b18, W19, b19):
    relu = jax.nn.relu
    z = jnp.zeros((3,), jnp.float32)
    out = relu(_conv(x, W11, b11, edge_index_3, selections_3, interps_3))
    out = out[clusters_2]
    out = relu(_conv(out, W12, b12, edge_index_2, selections_2, interps_2))
    out = relu(_conv(out, W13, b13, edge_index_2, selections_2, interps_2))
    out = relu(_conv(out, W14, b14, edge_index_2, selections_2, interps_2))
    out = relu(_conv(out, W15, b15, edge_index_2, selections_2, interps_2))
    out = out[clusters_1]
    out = relu(_conv(out, W16, b16, edge_index_1, selections_1, interps_1))
    out = relu(_conv(out, W17, b17, edge_index_1, selections_1, interps_1))
    out = out[clusters_0]
    out = relu(_conv(out, W18, b18, edge_index_0, selections_0, interps_0))
    out = _conv(out, W19, z, edge_index_0, selections_0, interps_0)
    bias = jnp.broadcast_to(b19, out.shape)
    return pl.pallas_call(
        _bias_add_kernel,
        grid=(64,),
        in_specs=[pl.BlockSpec((1024, 3), lambda i: (i, 0)),
                  pl.BlockSpec((1024, 3), lambda i: (i, 0))],
        out_specs=pl.BlockSpec((1024, 3), lambda i: (i, 0)),
        out_shape=jax.ShapeDtypeStruct(out.shape, out.dtype),
    )(out, bias)
